# baseline (speedup 1.0000x reference)
import functools

import jax
import jax.numpy as jnp
from jax import lax
from jax.experimental import pallas as pl
from jax.experimental.pallas import tpu as pltpu
from jax.experimental.pallas import tpu_sc as plsc

B = 16384
D = 64
NC = 2
NS = 16
NW = NC * NS
BPW = B // NW
CHUNK = 128
NCHUNK = BPW // CHUNK
L = 16
NGC = CHUNK // L


def _dot_body(user_hbm, item_hbm, ut_hbm, it_hbm,
              out_hbm,
              uidx_v, iidx_v, uq_v, iq_v, ubuf_v, ibuf_v, out_v, sem):
    wid = lax.axis_index("s") * NC + lax.axis_index("c")

    pltpu.sync_copy(user_hbm.at[wid], uidx_v)
    pltpu.sync_copy(item_hbm.at[wid], iidx_v)

    def halve(g, carry):
        sl = pl.ds(g * L, L)
        uq_v[sl] = lax.shift_right_logical(uidx_v[sl], 1)
        iq_v[sl] = lax.shift_right_logical(iidx_v[sl], 1)
        return carry

    lax.fori_loop(0, BPW // L, halve, 0)

    lane = lax.iota(jnp.int32, 16)
    dnums = lax.GatherDimensionNumbers(
        offset_dims=(), collapsed_slice_dims=(0,), start_index_map=(0,))

    def shuffle(x, idx):
        return lax.gather(x, idx[:, None], dnums, (1,),
                          mode=lax.GatherScatterMode.PROMISE_IN_BOUNDS)

    def lanesum(p):
        for sh in (8, 4, 2, 1):
            p = p + shuffle(p, lane ^ sh)
        return p

    for c in range(NCHUNK):
        csl = pl.ds(c * CHUNK, CHUNK)
        cu = pltpu.async_copy(ut_hbm.at[uq_v.at[csl]], ubuf_v, sem)
        ci = pltpu.async_copy(it_hbm.at[iq_v.at[csl]], ibuf_v, sem)
        cu.wait()
        ci.wait()

        def group(g, carry):
            base = c * CHUNK + g * L
            uvec = uidx_v[pl.ds(base, L)]
            ivec = iidx_v[pl.ds(base, L)]
            acc = jnp.zeros((L,), jnp.float32)
            for r in range(L):
                j = g * L + r
                uoff = (uvec[r] & 1) * D
                ioff = (ivec[r] & 1) * D
                p = (ubuf_v[j, pl.ds(uoff, 16)] * ibuf_v[j, pl.ds(ioff, 16)]
                     + ubuf_v[j, pl.ds(uoff + 16, 16)]
                     * ibuf_v[j, pl.ds(ioff + 16, 16)]
                     + ubuf_v[j, pl.ds(uoff + 32, 16)]
                     * ibuf_v[j, pl.ds(ioff + 32, 16)]
                     + ubuf_v[j, pl.ds(uoff + 48, 16)]
                     * ibuf_v[j, pl.ds(ioff + 48, 16)])
                acc = jnp.where(lane == r, lanesum(p), acc)
            out_v[pl.ds(base, L)] = acc
            return carry

        lax.fori_loop(0, NGC, group, 0)

    pltpu.sync_copy(out_v, out_hbm.at[pl.ds(wid * BPW, BPW)])


def _bias_body(user_hbm, item_hbm, dot_hbm, bu_hbm, bi_hbm, gb_hbm,
               out_hbm,
               uidx_v, iidx_v, ubias_v, ibias_v, dot_v, gb_v, sem):
    wid = lax.axis_index("s") * NC + lax.axis_index("c")

    pltpu.sync_copy(user_hbm.at[wid], uidx_v)
    pltpu.sync_copy(item_hbm.at[wid], iidx_v)
    pltpu.sync_copy(dot_hbm.at[pl.ds(wid * BPW, BPW)], dot_v)
    pltpu.sync_copy(gb_hbm, gb_v)

    copies = []
    for j in range(NCHUNK):
        sl = pl.ds(j * CHUNK, CHUNK)
        copies.append(pltpu.async_copy(bu_hbm.at[uidx_v.at[sl]],
                                       ubias_v.at[sl], sem))
        copies.append(pltpu.async_copy(bi_hbm.at[iidx_v.at[sl]],
                                       ibias_v.at[sl], sem))
    for cp in copies:
        cp.wait()

    gbv = gb_v[...]

    def finish(g, carry):
        sl = pl.ds(g * L, L)
        res = dot_v[sl] + gbv + ubias_v[sl] + ibias_v[sl]
        dot_v[sl] = jnp.minimum(jnp.maximum(res, 1.0), 5.0)
        return carry

    lax.fori_loop(0, BPW // L, finish, 0)

    pltpu.sync_copy(dot_v, out_hbm.at[pl.ds(wid * BPW, BPW)])


@jax.jit
def _svd_score(user_r, item_r, ut2, it2, bias_user_flat, bias_item_flat,
               gb16):
    mesh = plsc.VectorSubcoreMesh(core_axis_name="c", subcore_axis_name="s")
    dot_k = functools.partial(
        pl.kernel,
        out_type=jax.ShapeDtypeStruct((B,), jnp.float32),
        mesh=mesh,
        scratch_types=[
            pltpu.VMEM((BPW,), jnp.int32),
            pltpu.VMEM((BPW,), jnp.int32),
            pltpu.VMEM((BPW,), jnp.int32),
            pltpu.VMEM((BPW,), jnp.int32),
            pltpu.VMEM((CHUNK, 2 * D), jnp.float32),
            pltpu.VMEM((CHUNK, 2 * D), jnp.float32),
            pltpu.VMEM((BPW,), jnp.float32),
            pltpu.SemaphoreType.DMA,
        ],
        compiler_params=pltpu.CompilerParams(use_tc_tiling_on_sc=True),
    )(_dot_body)
    dot = dot_k(user_r, item_r, ut2, it2)

    bias_k = functools.partial(
        pl.kernel,
        out_type=jax.ShapeDtypeStruct((B,), jnp.float32),
        mesh=mesh,
        scratch_types=[
            pltpu.VMEM((BPW,), jnp.int32),
            pltpu.VMEM((BPW,), jnp.int32),
            pltpu.VMEM((BPW,), jnp.float32),
            pltpu.VMEM((BPW,), jnp.float32),
            pltpu.VMEM((BPW,), jnp.float32),
            pltpu.VMEM((16,), jnp.float32),
            pltpu.SemaphoreType.DMA,
        ],
        compiler_params=pltpu.CompilerParams(use_tc_tiling_on_sc=False),
    )(_bias_body)
    return bias_k(user_r, item_r, dot, bias_user_flat, bias_item_flat, gb16)


def kernel(user, item, user_table, item_table, bias_user_table,
           bias_item_table, global_bias):
    gb16 = jnp.broadcast_to(
        jnp.asarray(global_bias, jnp.float32).reshape(1), (16,))
    out = _svd_score(user.reshape(NW, BPW), item.reshape(NW, BPW),
                     user_table.reshape(500000, 128),
                     item_table.reshape(500000, 128),
                     bias_user_table.reshape(-1), bias_item_table.reshape(-1),
                     gb16)
    return out.reshape(1, B)


# speedup vs baseline: 1.1962x; 1.1962x over previous
import functools

import jax
import jax.numpy as jnp
from jax import lax
from jax.experimental import pallas as pl
from jax.experimental.pallas import tpu as pltpu
from jax.experimental.pallas import tpu_sc as plsc

B = 16384
D = 64
NC = 2
NS = 16
NW = NC * NS
BPW = B // NW
CHUNK = 128
NCHUNK = BPW // CHUNK
L = 16
NGC = CHUNK // L


def _dot_body(user_hbm, item_hbm, tab_hbm,
              out_hbm,
              uidx_v, iidx_v, ubuf_v, ibuf_v, out_v, sem):
    wid = lax.axis_index("s") * NC + lax.axis_index("c")

    pltpu.sync_copy(user_hbm.at[wid], uidx_v)
    pltpu.sync_copy(item_hbm.at[wid], iidx_v)

    lane = lax.iota(jnp.int32, 16)
    dnums = lax.GatherDimensionNumbers(
        offset_dims=(), collapsed_slice_dims=(0,), start_index_map=(0,))

    def shuffle(x, idx):
        return lax.gather(x, idx[:, None], dnums, (1,),
                          mode=lax.GatherScatterMode.PROMISE_IN_BOUNDS)

    def lanesum(p):
        for sh in (8, 4, 2, 1):
            p = p + shuffle(p, lane ^ sh)
        return p

    for c in range(NCHUNK):
        csl = pl.ds(c * CHUNK, CHUNK)
        cu = pltpu.async_copy(tab_hbm.at[uidx_v.at[csl]], ubuf_v, sem)
        ci = pltpu.async_copy(tab_hbm.at[iidx_v.at[csl]], ibuf_v, sem)
        cu.wait()
        ci.wait()

        def group(g, carry):
            acc = jnp.zeros((L,), jnp.float32)
            for r in range(L):
                j = g * L + r
                p = (ubuf_v[j, pl.ds(0, 16)] * ibuf_v[j, pl.ds(D, 16)]
                     + ubuf_v[j, pl.ds(16, 16)] * ibuf_v[j, pl.ds(D + 16, 16)]
                     + ubuf_v[j, pl.ds(32, 16)] * ibuf_v[j, pl.ds(D + 32, 16)]
                     + ubuf_v[j, pl.ds(48, 16)] * ibuf_v[j, pl.ds(D + 48, 16)])
                acc = jnp.where(lane == r, lanesum(p), acc)
            out_v[pl.ds(c * CHUNK + g * L, L)] = acc
            return carry

        lax.fori_loop(0, NGC, group, 0)

    pltpu.sync_copy(out_v, out_hbm.at[pl.ds(wid * BPW, BPW)])


def _bias_body(user_hbm, item_hbm, dot_hbm, bu_hbm, bi_hbm, gb_hbm,
               out_hbm,
               uidx_v, iidx_v, ubias_v, ibias_v, dot_v, gb_v, sem):
    wid = lax.axis_index("s") * NC + lax.axis_index("c")

    pltpu.sync_copy(user_hbm.at[wid], uidx_v)
    pltpu.sync_copy(item_hbm.at[wid], iidx_v)
    pltpu.sync_copy(dot_hbm.at[pl.ds(wid * BPW, BPW)], dot_v)
    pltpu.sync_copy(gb_hbm, gb_v)

    copies = []
    for j in range(NCHUNK):
        sl = pl.ds(j * CHUNK, CHUNK)
        copies.append(pltpu.async_copy(bu_hbm.at[uidx_v.at[sl]],
                                       ubias_v.at[sl], sem))
        copies.append(pltpu.async_copy(bi_hbm.at[iidx_v.at[sl]],
                                       ibias_v.at[sl], sem))
    for cp in copies:
        cp.wait()

    gbv = gb_v[...]

    def finish(g, carry):
        sl = pl.ds(g * L, L)
        res = dot_v[sl] + gbv + ubias_v[sl] + ibias_v[sl]
        dot_v[sl] = jnp.minimum(jnp.maximum(res, 1.0), 5.0)
        return carry

    lax.fori_loop(0, BPW // L, finish, 0)

    pltpu.sync_copy(dot_v, out_hbm.at[pl.ds(wid * BPW, BPW)])


@jax.jit
def _svd_score(user_r, item_r, table2, bias_user_flat, bias_item_flat,
               gb16):
    mesh = plsc.VectorSubcoreMesh(core_axis_name="c", subcore_axis_name="s")
    dot_k = functools.partial(
        pl.kernel,
        out_type=jax.ShapeDtypeStruct((B,), jnp.float32),
        mesh=mesh,
        scratch_types=[
            pltpu.VMEM((BPW,), jnp.int32),
            pltpu.VMEM((BPW,), jnp.int32),
            pltpu.VMEM((CHUNK, 2 * D), jnp.float32),
            pltpu.VMEM((CHUNK, 2 * D), jnp.float32),
            pltpu.VMEM((BPW,), jnp.float32),
            pltpu.SemaphoreType.DMA,
        ],
        compiler_params=pltpu.CompilerParams(use_tc_tiling_on_sc=True),
    )(_dot_body)
    dot = dot_k(user_r, item_r, table2)

    bias_k = functools.partial(
        pl.kernel,
        out_type=jax.ShapeDtypeStruct((B,), jnp.float32),
        mesh=mesh,
        scratch_types=[
            pltpu.VMEM((BPW,), jnp.int32),
            pltpu.VMEM((BPW,), jnp.int32),
            pltpu.VMEM((BPW,), jnp.float32),
            pltpu.VMEM((BPW,), jnp.float32),
            pltpu.VMEM((BPW,), jnp.float32),
            pltpu.VMEM((16,), jnp.float32),
            pltpu.SemaphoreType.DMA,
        ],
        compiler_params=pltpu.CompilerParams(use_tc_tiling_on_sc=False),
    )(_bias_body)
    return bias_k(user_r, item_r, dot, bias_user_flat, bias_item_flat, gb16)


def kernel(user, item, user_table, item_table, bias_user_table,
           bias_item_table, global_bias):
    gb16 = jnp.broadcast_to(
        jnp.asarray(global_bias, jnp.float32).reshape(1), (16,))
    table2 = jnp.concatenate([user_table, item_table], axis=1)
    out = _svd_score(user.reshape(NW, BPW), item.reshape(NW, BPW), table2,
                     bias_user_table.reshape(-1), bias_item_table.reshape(-1),
                     gb16)
    return out.reshape(1, B)


# speedup vs baseline: 3.1551x; 2.6377x over previous
import functools

import jax
import jax.numpy as jnp
from jax import lax
from jax.experimental import pallas as pl
from jax.experimental.pallas import tpu as pltpu
from jax.experimental.pallas import tpu_sc as plsc

B = 16384
D = 64
NC = 2
NS = 16
NW = NC * NS
BPW = B // NW
CHUNK = 128
NCHUNK = BPW // CHUNK
L = 16
NG = BPW // L
RING = 4
BLK = 32 * 1024


def _dot_body(user_hbm, item_hbm, ut_hbm, it_hbm,
              out_hbm,
              uidx_v, iidx_v, ubufs_v, ibufs_v, out_v, sem):
    wid = lax.axis_index("s") * NC + lax.axis_index("c")

    pltpu.sync_copy(user_hbm.at[wid], uidx_v.at[pl.ds(0, BPW)])
    pltpu.sync_copy(item_hbm.at[wid], iidx_v.at[pl.ds(0, BPW)])

    lane = lax.iota(jnp.int32, 16)
    dnums = lax.GatherDimensionNumbers(
        offset_dims=(), collapsed_slice_dims=(0,), start_index_map=(0,))

    def shuffle(x, idx):
        return lax.gather(x, idx[:, None], dnums, (1,),
                          mode=lax.GatherScatterMode.PROMISE_IN_BOUNDS)

    def fire(uvec, ivec, r, slot):
        ublk = pl.multiple_of((uvec[r] >> 7) * 128, 128)
        iblk = pl.multiple_of((ivec[r] >> 7) * 128, 128)
        pltpu.async_copy(ut_hbm.at[:, pl.ds(ublk, 128)],
                         ubufs_v.at[slot], sem)
        pltpu.async_copy(it_hbm.at[:, pl.ds(iblk, 128)],
                         ibufs_v.at[slot], sem)

    def drain(slot):
        pltpu.make_async_copy(ut_hbm.at[:, pl.ds(0, 128)],
                              ubufs_v.at[slot], sem).wait()
        pltpu.make_async_copy(it_hbm.at[:, pl.ds(0, 128)],
                              ibufs_v.at[slot], sem).wait()

    def process(uvec, ivec, r, slot, acc):
        drain(slot)
        cu = uvec[r] & 127
        ci = ivec[r] & 127
        cu_al = cu & ~15
        ci_al = ci & ~15
        ulane = jnp.broadcast_to(cu & 15, (L,))
        ilane = jnp.broadcast_to(ci & 15, (L,))

        def dstep(k, p):
            d = k * 8
            for dd in range(8):
                u16 = ubufs_v[slot, d + dd, pl.ds(cu_al, 16)]
                i16 = ibufs_v[slot, d + dd, pl.ds(ci_al, 16)]
                p = p + shuffle(u16, ulane) * shuffle(i16, ilane)
            return p

        p = lax.fori_loop(0, D // 8, dstep, jnp.zeros((L,), jnp.float32))
        return jnp.where(lane == r, p, acc)

    uvec0 = uidx_v[pl.ds(0, L)]
    ivec0 = iidx_v[pl.ds(0, L)]
    for r in range(RING):
        fire(uvec0, ivec0, r, r)

    def group(g, carry):
        uvec, ivec = carry
        unext = uidx_v[pl.ds((g + 1) * L, L)]
        inext = iidx_v[pl.ds((g + 1) * L, L)]
        acc = jnp.zeros((L,), jnp.float32)
        for r in range(L):
            acc = process(uvec, ivec, r, r % RING, acc)
            if r < L - RING:
                fire(uvec, ivec, r + RING, (r + RING) % RING)
            else:
                @pl.when(g < NG - 1)
                def _():
                    fire(unext, inext, r + RING - L, (r + RING) % RING)
        out_v[pl.ds(g * L, L)] = acc
        return (unext, inext)

    lax.fori_loop(0, NG, group, (uvec0, ivec0))

    pltpu.sync_copy(out_v, out_hbm.at[pl.ds(wid * BPW, BPW)])


def _bias_body(user_hbm, item_hbm, dot_hbm, bu_hbm, bi_hbm, gb_hbm,
               out_hbm,
               uidx_v, iidx_v, ubias_v, ibias_v, dot_v, gb_v, sem):
    wid = lax.axis_index("s") * NC + lax.axis_index("c")

    pltpu.sync_copy(user_hbm.at[wid], uidx_v)
    pltpu.sync_copy(item_hbm.at[wid], iidx_v)
    pltpu.sync_copy(dot_hbm.at[pl.ds(wid * BPW, BPW)], dot_v)
    pltpu.sync_copy(gb_hbm, gb_v)

    copies = []
    for j in range(NCHUNK):
        sl = pl.ds(j * CHUNK, CHUNK)
        copies.append(pltpu.async_copy(bu_hbm.at[uidx_v.at[sl]],
                                       ubias_v.at[sl], sem))
        copies.append(pltpu.async_copy(bi_hbm.at[iidx_v.at[sl]],
                                       ibias_v.at[sl], sem))
    for cp in copies:
        cp.wait()

    gbv = gb_v[...]

    def finish(g, carry):
        sl = pl.ds(g * L, L)
        res = dot_v[sl] + gbv + ubias_v[sl] + ibias_v[sl]
        dot_v[sl] = jnp.minimum(jnp.maximum(res, 1.0), 5.0)
        return carry

    lax.fori_loop(0, BPW // L, finish, 0)

    pltpu.sync_copy(dot_v, out_hbm.at[pl.ds(wid * BPW, BPW)])


@jax.jit
def _svd_score(user_r, item_r, ut_t, it_t, bias_user_flat, bias_item_flat,
               gb16):
    mesh = plsc.VectorSubcoreMesh(core_axis_name="c", subcore_axis_name="s")
    dot_k = functools.partial(
        pl.kernel,
        out_type=jax.ShapeDtypeStruct((B,), jnp.float32),
        mesh=mesh,
        scratch_types=[
            pltpu.VMEM((BPW + L,), jnp.int32),
            pltpu.VMEM((BPW + L,), jnp.int32),
            pltpu.VMEM((RING, D, 128), jnp.float32),
            pltpu.VMEM((RING, D, 128), jnp.float32),
            pltpu.VMEM((BPW,), jnp.float32),
            pltpu.SemaphoreType.DMA,
        ],
        compiler_params=pltpu.CompilerParams(use_tc_tiling_on_sc=True),
    )(_dot_body)
    dot = dot_k(user_r, item_r, ut_t, it_t)

    bias_k = functools.partial(
        pl.kernel,
        out_type=jax.ShapeDtypeStruct((B,), jnp.float32),
        mesh=mesh,
        scratch_types=[
            pltpu.VMEM((BPW,), jnp.int32),
            pltpu.VMEM((BPW,), jnp.int32),
            pltpu.VMEM((BPW,), jnp.float32),
            pltpu.VMEM((BPW,), jnp.float32),
            pltpu.VMEM((BPW,), jnp.float32),
            pltpu.VMEM((16,), jnp.float32),
            pltpu.SemaphoreType.DMA,
        ],
        compiler_params=pltpu.CompilerParams(use_tc_tiling_on_sc=False),
    )(_bias_body)
    return bias_k(user_r, item_r, dot, bias_user_flat, bias_item_flat, gb16)


def kernel(user, item, user_table, item_table, bias_user_table,
           bias_item_table, global_bias):
    gb16 = jnp.broadcast_to(
        jnp.asarray(global_bias, jnp.float32).reshape(1), (16,))
    out = _svd_score(user.reshape(NW, BPW), item.reshape(NW, BPW),
                     user_table.T, item_table.T,
                     bias_user_table.reshape(-1), bias_item_table.reshape(-1),
                     gb16)
    return out.reshape(1, B)
